# baseline (speedup 1.0000x reference)
import jax
import jax.numpy as jnp
from jax.experimental import pallas as pl
from jax.experimental.pallas import tpu as pltpu

_N_EDGES = 8192
_N_NODES = 2048
_N_TRI = 4096
_F = 128

_STEPS = 16
_JB = _N_TRI // _STEPS
_IB = _N_NODES // _STEPS


def _scone_kernel(x_ref, w0_ref, w1_ref, w2_ref, b2_ref, b1_ref, out_ref,
                  xw0_s, xw2_s):
    s = pl.program_id(0)
    n_steps = pl.num_programs(0)

    @pl.when(s == 0)
    def _():
        xb = x_ref[...].astype(jnp.bfloat16)
        xw0_s[...] = jnp.dot(xb, w0_ref[...].astype(jnp.bfloat16),
                             preferred_element_type=jnp.float32).astype(jnp.bfloat16)
        xw2_s[...] = jnp.dot(xb, w2_ref[...].astype(jnp.bfloat16),
                             preferred_element_type=jnp.float32).astype(jnp.bfloat16)
        out_ref[...] = jnp.dot(xb, w1_ref[...].astype(jnp.bfloat16),
                               preferred_element_type=jnp.float32)

    b2 = b2_ref[...].astype(jnp.bfloat16)
    t = jax.lax.dot_general(b2, xw2_s[...], (((0,), (0,)), ((), ())),
                            preferred_element_type=jnp.float32)
    d = jnp.dot(b2, t.astype(jnp.bfloat16), preferred_element_type=jnp.float32)

    b1 = b1_ref[...].astype(jnp.bfloat16)
    n = jnp.dot(b1, xw0_s[...], preferred_element_type=jnp.float32)
    e = jax.lax.dot_general(b1, n.astype(jnp.bfloat16), (((0,), (0,)), ((), ())),
                            preferred_element_type=jnp.float32)

    @pl.when(s != n_steps - 1)
    def _():
        out_ref[...] += d + e

    @pl.when(s == n_steps - 1)
    def _():
        out_ref[...] = jnp.maximum(out_ref[...] + (d + e), 0.0)


def kernel(x, B1, B2, W0, W1, W2):
    return pl.pallas_call(
        _scone_kernel,
        grid=(_STEPS,),
        in_specs=[
            pl.BlockSpec((_N_EDGES, _F), lambda s: (0, 0)),
            pl.BlockSpec((_F, _F), lambda s: (0, 0)),
            pl.BlockSpec((_F, _F), lambda s: (0, 0)),
            pl.BlockSpec((_F, _F), lambda s: (0, 0)),
            pl.BlockSpec((_N_EDGES, _JB), lambda s: (0, s)),
            pl.BlockSpec((_IB, _N_EDGES), lambda s: (s, 0)),
        ],
        out_specs=pl.BlockSpec((_N_EDGES, _F), lambda s: (0, 0)),
        out_shape=jax.ShapeDtypeStruct((_N_EDGES, _F), jnp.float32),
        scratch_shapes=[
            pltpu.VMEM((_N_EDGES, _F), jnp.bfloat16),
            pltpu.VMEM((_N_EDGES, _F), jnp.bfloat16),
        ],
        compiler_params=pltpu.CompilerParams(
            dimension_semantics=("arbitrary",)),
    )(x, W0, W1, W2, B2, B1)


# speedup vs baseline: 1.0529x; 1.0529x over previous
import jax
import jax.numpy as jnp
from jax.experimental import pallas as pl
from jax.experimental.pallas import tpu as pltpu

_N_EDGES = 8192
_N_NODES = 2048
_N_TRI = 4096
_F = 128

_STEPS = 16
_JB = _N_TRI // _STEPS
_IB = _N_NODES // _STEPS


def _scone_kernel(x_ref, w0_ref, w1_ref, w2_ref, b2a_ref, b2b_ref, b1_ref,
                  out_ref, xw0_s, xw2_s):
    s = pl.program_id(0)
    n_steps = pl.num_programs(0)

    @pl.when(s == 0)
    def _():
        xb = x_ref[...].astype(jnp.bfloat16)
        xw0_s[...] = jnp.dot(xb, w0_ref[...].astype(jnp.bfloat16),
                             preferred_element_type=jnp.float32).astype(jnp.bfloat16)
        xw2_s[...] = jnp.dot(xb, w2_ref[...].astype(jnp.bfloat16),
                             preferred_element_type=jnp.float32).astype(jnp.bfloat16)
        out_ref[...] = jnp.dot(xb, w1_ref[...].astype(jnp.bfloat16),
                               preferred_element_type=jnp.float32)

    half = _N_EDGES // 2
    b2a = b2a_ref[...].astype(jnp.bfloat16)
    b2b = b2b_ref[...].astype(jnp.bfloat16)
    t = (jax.lax.dot_general(b2a, xw2_s[:half, :], (((0,), (0,)), ((), ())),
                             preferred_element_type=jnp.float32)
         + jax.lax.dot_general(b2b, xw2_s[half:, :], (((0,), (0,)), ((), ())),
                               preferred_element_type=jnp.float32))
    tb = t.astype(jnp.bfloat16)
    da = jnp.dot(b2a, tb, preferred_element_type=jnp.float32)
    db = jnp.dot(b2b, tb, preferred_element_type=jnp.float32)

    b1 = b1_ref[...].astype(jnp.bfloat16)
    n = jnp.dot(b1, xw0_s[...], preferred_element_type=jnp.float32)
    e = jax.lax.dot_general(b1, n.astype(jnp.bfloat16), (((0,), (0,)), ((), ())),
                            preferred_element_type=jnp.float32)

    out_ref[:half, :] += da + e[:half, :]
    out_ref[half:, :] += db + e[half:, :]

    @pl.when(s == n_steps - 1)
    def _():
        out_ref[...] = jnp.maximum(out_ref[...], 0.0)


def kernel(x, B1, B2, W0, W1, W2):
    return pl.pallas_call(
        _scone_kernel,
        grid=(_STEPS,),
        in_specs=[
            pl.BlockSpec((_N_EDGES, _F), lambda s: (0, 0)),
            pl.BlockSpec((_F, _F), lambda s: (0, 0)),
            pl.BlockSpec((_F, _F), lambda s: (0, 0)),
            pl.BlockSpec((_F, _F), lambda s: (0, 0)),
            pl.BlockSpec((_N_EDGES // 2, _JB), lambda s: (0, s)),
            pl.BlockSpec((_N_EDGES // 2, _JB), lambda s: (1, s)),
            pl.BlockSpec((_IB, _N_EDGES), lambda s: (s, 0)),
        ],
        out_specs=pl.BlockSpec((_N_EDGES, _F), lambda s: (0, 0)),
        out_shape=jax.ShapeDtypeStruct((_N_EDGES, _F), jnp.float32),
        scratch_shapes=[
            pltpu.VMEM((_N_EDGES, _F), jnp.bfloat16),
            pltpu.VMEM((_N_EDGES, _F), jnp.bfloat16),
        ],
        compiler_params=pltpu.CompilerParams(
            dimension_semantics=("arbitrary",)),
    )(x, W0, W1, W2, B2, B2, B1)


# speedup vs baseline: 1.0578x; 1.0046x over previous
import jax
import jax.numpy as jnp
from jax.experimental import pallas as pl
from jax.experimental.pallas import tpu as pltpu

_N_EDGES = 8192
_N_NODES = 2048
_N_TRI = 4096
_F = 128

_STEPS = 16
_JB = _N_TRI // _STEPS
_IB = _N_NODES // _STEPS


def _scone_kernel(x_ref, w0_ref, w1_ref, w2_ref, b2_ref, b1_ref, out_ref,
                  xw0_s, xw2_s):
    s = pl.program_id(0)
    n_steps = pl.num_programs(0)

    @pl.when(s == 0)
    def _():
        xb = x_ref[...].astype(jnp.bfloat16)
        xw0_s[...] = jnp.dot(xb, w0_ref[...].astype(jnp.bfloat16),
                             preferred_element_type=jnp.float32).astype(jnp.bfloat16)
        xw2_s[...] = jnp.dot(xb, w2_ref[...].astype(jnp.bfloat16),
                             preferred_element_type=jnp.float32).astype(jnp.bfloat16)
        out_ref[...] = jnp.dot(xb, w1_ref[...].astype(jnp.bfloat16),
                               preferred_element_type=jnp.float32)

    b2 = b2_ref[...].astype(jnp.bfloat16)
    t = jax.lax.dot_general(b2, xw2_s[...], (((0,), (0,)), ((), ())),
                            preferred_element_type=jnp.float32)
    d = jnp.dot(b2, t.astype(jnp.bfloat16), preferred_element_type=jnp.float32)

    b1 = b1_ref[...].astype(jnp.bfloat16)
    n = jnp.dot(b1, xw0_s[...], preferred_element_type=jnp.float32)
    e = jax.lax.dot_general(b1, n.astype(jnp.bfloat16), (((0,), (0,)), ((), ())),
                            preferred_element_type=jnp.float32)

    out_ref[...] += d + e

    @pl.when(s == n_steps - 1)
    def _():
        out_ref[...] = jnp.maximum(out_ref[...], 0.0)


def kernel(x, B1, B2, W0, W1, W2):
    return pl.pallas_call(
        _scone_kernel,
        grid=(_STEPS,),
        in_specs=[
            pl.BlockSpec((_N_EDGES, _F), lambda s: (0, 0)),
            pl.BlockSpec((_F, _F), lambda s: (0, 0)),
            pl.BlockSpec((_F, _F), lambda s: (0, 0)),
            pl.BlockSpec((_F, _F), lambda s: (0, 0)),
            pl.BlockSpec((_N_EDGES, _JB), lambda s: (0, s)),
            pl.BlockSpec((_IB, _N_EDGES), lambda s: (s, 0)),
        ],
        out_specs=pl.BlockSpec((_N_EDGES, _F), lambda s: (0, 0)),
        out_shape=jax.ShapeDtypeStruct((_N_EDGES, _F), jnp.float32),
        scratch_shapes=[
            pltpu.VMEM((_N_EDGES, _F), jnp.bfloat16),
            pltpu.VMEM((_N_EDGES, _F), jnp.bfloat16),
        ],
        compiler_params=pltpu.CompilerParams(
            dimension_semantics=("arbitrary",)),
    )(x, W0, W1, W2, B2, B1)
